# baseline (speedup 1.0000x reference)
import functools

import jax
import jax.numpy as jnp
from jax import lax
from jax.experimental import pallas as pl
from jax.experimental.pallas import tpu as pltpu
from jax.experimental.pallas import tpu_sc as plsc

N = 10000
NP = 10240
E = 320000
H = 64
L = 6
G = 16
NW = 32
CE = 8000
KT = 16000
CN = 2048
EB = E // NW

_NEG_INF = float("-inf")


def _mesh():
    return plsc.VectorSubcoreMesh(core_axis_name="c", subcore_axis_name="s")


def _wid():
    return lax.axis_index("s") * 2 + lax.axis_index("c")


@functools.partial(
    pl.kernel, mesh=_mesh(),
    compiler_params=pltpu.CompilerParams(needs_layout_passes=False),
    out_type=jax.ShapeDtypeStruct((8 * E,), jnp.float32),
    scratch_types=[
        pltpu.VMEM((NP,), jnp.float32), pltpu.VMEM((NP,), jnp.float32),
        pltpu.VMEM((NP,), jnp.float32),
        pltpu.VMEM((EB,), jnp.int32), pltpu.VMEM((EB,), jnp.int32),
        pltpu.VMEM((EB,), jnp.float32), pltpu.VMEM((EB,), jnp.float32),
        pltpu.VMEM((EB,), jnp.float32),
    ],
)
def _rel_kernel(posT_hbm, src_hbm, dst_hbm, out_hbm,
                px, py, pz, srcb, dstb, rx, ry, rz):
    w = _wid()
    base = w * EB
    pltpu.sync_copy(posT_hbm.at[pl.ds(0, NP)], px)
    pltpu.sync_copy(posT_hbm.at[pl.ds(NP, NP)], py)
    pltpu.sync_copy(posT_hbm.at[pl.ds(2 * NP, NP)], pz)
    pltpu.sync_copy(src_hbm.at[pl.ds(base, EB)], srcb)
    pltpu.sync_copy(dst_hbm.at[pl.ds(base, EB)], dstb)

    def body(g, _):
        sl = pl.ds(g * 16, 16)
        si = srcb[sl]
        di = dstb[sl]
        rx[sl] = plsc.load_gather(px, [si]) - plsc.load_gather(px, [di])
        ry[sl] = plsc.load_gather(py, [si]) - plsc.load_gather(py, [di])
        rz[sl] = plsc.load_gather(pz, [si]) - plsc.load_gather(pz, [di])
        return 0

    lax.fori_loop(0, EB // 16, body, 0)
    pltpu.sync_copy(rx, out_hbm.at[pl.ds(0 * E + base, EB)])
    pltpu.sync_copy(ry, out_hbm.at[pl.ds(1 * E + base, EB)])
    pltpu.sync_copy(rz, out_hbm.at[pl.ds(2 * E + base, EB)])

    def zbody(i, _):
        rx[pl.ds(i * 16, 16)] = jnp.zeros((16,), jnp.float32)
        return 0

    lax.fori_loop(0, EB // 16, zbody, 0)
    for r in range(3, 8):
        pltpu.sync_copy(rx, out_hbm.at[pl.ds(r * E + base, EB)])


@functools.partial(
    pl.kernel, mesh=_mesh(),
    compiler_params=pltpu.CompilerParams(needs_layout_passes=False),
    out_type=jax.ShapeDtypeStruct((H * E,), jnp.float32),
    scratch_types=[
        pltpu.VMEM((NP,), jnp.float32), pltpu.VMEM((NP,), jnp.float32),
        pltpu.VMEM((NP,), jnp.float32), pltpu.VMEM((NP,), jnp.float32),
        pltpu.VMEM((CE,), jnp.int32), pltpu.VMEM((CE,), jnp.int32),
        pltpu.VMEM((CE,), jnp.int32), pltpu.VMEM((CE,), jnp.int32),
        pltpu.VMEM((CE,), jnp.float32), pltpu.VMEM((CE,), jnp.float32),
        pltpu.VMEM((CE,), jnp.float32), pltpu.VMEM((CE,), jnp.float32),
        pltpu.SemaphoreType.DMA, pltpu.SemaphoreType.DMA,
        pltpu.SemaphoreType.DMA, pltpu.SemaphoreType.DMA,
    ],
)
def _gather_kernel(uvT_hbm, src_hbm, dst_hbm, out_hbm,
                   u0, u1, v0, v1, srcb0, dstb0, srcb1, dstb1,
                   z00, z10, z01, z11, isem0, isem1, osem0, osem1):
    w = _wid()
    f0 = 2 * w
    srcbs = (srcb0, srcb1)
    dstbs = (dstb0, dstb1)
    z0s = (z00, z01)
    z1s = (z10, z11)
    isems = (isem0, isem1)
    osems = (osem0, osem1)
    NC = E // CE

    for b in range(2):
        pltpu.async_copy(src_hbm.at[pl.ds(b * CE, CE)], srcbs[b], isems[b])
        pltpu.async_copy(dst_hbm.at[pl.ds(b * CE, CE)], dstbs[b], isems[b])
    pltpu.sync_copy(uvT_hbm.at[pl.ds(f0 * NP, NP)], u0)
    pltpu.sync_copy(uvT_hbm.at[pl.ds((f0 + 1) * NP, NP)], u1)
    pltpu.sync_copy(uvT_hbm.at[pl.ds((H + f0) * NP, NP)], v0)
    pltpu.sync_copy(uvT_hbm.at[pl.ds((H + f0 + 1) * NP, NP)], v1)

    def outer(i, _):
        for b in range(2):
            c = i * 2 + b
            eb = c * CE
            srcb, dstb, z0, z1 = srcbs[b], dstbs[b], z0s[b], z1s[b]
            pltpu.make_async_copy(src_hbm.at[pl.ds(eb, CE)], srcb,
                                  isems[b]).wait()
            pltpu.make_async_copy(dst_hbm.at[pl.ds(eb, CE)], dstb,
                                  isems[b]).wait()

            @pl.when(c >= 2)
            def _():
                pltpu.make_async_copy(
                    z0, out_hbm.at[pl.ds(f0 * E + eb, CE)], osems[b]).wait()
                pltpu.make_async_copy(
                    z1, out_hbm.at[pl.ds((f0 + 1) * E + eb, CE)],
                    osems[b]).wait()

            def body(g, _):
                for k in range(4):
                    sl = pl.ds(g * 64 + k * 16, 16)
                    si = srcb[sl]
                    di = dstb[sl]
                    z0[sl] = (plsc.load_gather(u0, [di])
                              + plsc.load_gather(v0, [si]))
                    z1[sl] = (plsc.load_gather(u1, [di])
                              + plsc.load_gather(v1, [si]))
                return 0

            lax.fori_loop(0, CE // 64, body, 0)
            pltpu.async_copy(z0, out_hbm.at[pl.ds(f0 * E + eb, CE)], osems[b])
            pltpu.async_copy(z1, out_hbm.at[pl.ds((f0 + 1) * E + eb, CE)],
                             osems[b])

            @pl.when(c + 2 < NC)
            def _():
                eb2 = eb + 2 * CE
                pltpu.async_copy(src_hbm.at[pl.ds(eb2, CE)], srcb, isems[b])
                pltpu.async_copy(dst_hbm.at[pl.ds(eb2, CE)], dstb, isems[b])
        return 0

    lax.fori_loop(0, NC // 2, outer, 0)
    for b in range(2):
        eb = (NC - 2 + b) * CE
        pltpu.make_async_copy(
            z0s[b], out_hbm.at[pl.ds(f0 * E + eb, CE)], osems[b]).wait()
        pltpu.make_async_copy(
            z1s[b], out_hbm.at[pl.ds((f0 + 1) * E + eb, CE)], osems[b]).wait()


@functools.partial(
    pl.kernel, mesh=_mesh(),
    compiler_params=pltpu.CompilerParams(needs_layout_passes=False),
    out_type=jax.ShapeDtypeStruct((H * NP,), jnp.float32),
    scratch_types=[
        pltpu.VMEM((NP,), jnp.float32), pltpu.VMEM((NP,), jnp.float32),
        pltpu.VMEM((CE,), jnp.int32), pltpu.VMEM((CE,), jnp.int32),
        pltpu.VMEM((CE,), jnp.float32), pltpu.VMEM((CE,), jnp.float32),
        pltpu.VMEM((CE,), jnp.float32), pltpu.VMEM((CE,), jnp.float32),
        pltpu.SemaphoreType.DMA, pltpu.SemaphoreType.DMA,
    ],
)
def _scatter_kernel(mT_hbm, dst_hbm, out_hbm, acc0, acc1,
                    dstb0, dstb1, m0b0, m1b0, m0b1, m1b1, isem0, isem1):
    w = _wid()
    f0 = 2 * w
    dstbs = (dstb0, dstb1)
    m0bs = (m0b0, m0b1)
    m1bs = (m1b0, m1b1)
    isems = (isem0, isem1)
    NC = E // CE

    for b in range(2):
        eb = b * CE
        pltpu.async_copy(dst_hbm.at[pl.ds(eb, CE)], dstbs[b], isems[b])
        pltpu.async_copy(mT_hbm.at[pl.ds(f0 * E + eb, CE)], m0bs[b], isems[b])
        pltpu.async_copy(mT_hbm.at[pl.ds((f0 + 1) * E + eb, CE)], m1bs[b],
                         isems[b])

    def init(i, _):
        sl = pl.ds(i * 16, 16)
        acc0[sl] = jnp.full((16,), _NEG_INF, jnp.float32)
        acc1[sl] = jnp.full((16,), _NEG_INF, jnp.float32)
        return 0

    lax.fori_loop(0, NP // 16, init, 0)

    def outer_sc(i, _):
      for b in range(2):
        c = i * 2 + b
        eb = c * CE
        dstb, m0b, m1b = dstbs[b], m0bs[b], m1bs[b]
        pltpu.make_async_copy(dst_hbm.at[pl.ds(eb, CE)], dstb, isems[b]).wait()
        pltpu.make_async_copy(mT_hbm.at[pl.ds(f0 * E + eb, CE)], m0b,
                              isems[b]).wait()
        pltpu.make_async_copy(mT_hbm.at[pl.ds((f0 + 1) * E + eb, CE)], m1b,
                              isems[b]).wait()

        def body(g, _):
            for k in range(2):
                sl = pl.ds(g * 32 + k * 16, 16)
                idx = dstb[sl]
                val0 = m0b[sl]
                val1 = m1b[sl]
                cur0 = plsc.load_gather(acc0, [idx])
                cur1 = plsc.load_gather(acc1, [idx])
                plsc.store_scatter(acc0, [idx], jnp.maximum(cur0, val0))
                plsc.store_scatter(acc1, [idx], jnp.maximum(cur1, val1))
                c0 = plsc.load_gather(acc0, [idx])
                c1 = plsc.load_gather(acc1, [idx])

                def w_cond(nd):
                    n0, n1 = nd
                    return jnp.sum((n0 | n1).astype(jnp.int32)) > 0

                def w_body(nd):
                    n0, n1 = nd
                    plsc.store_scatter(acc0, [idx], val0, mask=n0)
                    plsc.store_scatter(acc1, [idx], val1, mask=n1)
                    r0 = plsc.load_gather(acc0, [idx])
                    r1 = plsc.load_gather(acc1, [idx])
                    return (val0 > r0, val1 > r1)

                lax.while_loop(w_cond, w_body, (val0 > c0, val1 > c1))
            return 0

        lax.fori_loop(0, CE // 32, body, 0)

        @pl.when(c + 2 < NC)
        def _():
            eb2 = eb + 2 * CE
            pltpu.async_copy(dst_hbm.at[pl.ds(eb2, CE)], dstb, isems[b])
            pltpu.async_copy(mT_hbm.at[pl.ds(f0 * E + eb2, CE)], m0b,
                             isems[b])
            pltpu.async_copy(mT_hbm.at[pl.ds((f0 + 1) * E + eb2, CE)], m1b,
                             isems[b])
      return 0

    lax.fori_loop(0, NC // 2, outer_sc, 0)

    def fini(i, _):
        sl = pl.ds(i * 16, 16)
        a0 = acc0[sl]
        a1 = acc1[sl]
        acc0[sl] = jnp.where(a0 == _NEG_INF, jnp.float32(0.0), a0)
        acc1[sl] = jnp.where(a1 == _NEG_INF, jnp.float32(0.0), a1)
        return 0

    lax.fori_loop(0, NP // 16, fini, 0)
    pltpu.sync_copy(acc0, out_hbm.at[pl.ds(f0 * NP, NP)])
    pltpu.sync_copy(acc1, out_hbm.at[pl.ds((f0 + 1) * NP, NP)])


def _enc_body(xT_ref, w0t_ref, b0_ref, w1t_ref, b1_ref, out_ref):
    a = jnp.dot(w0t_ref[...], xT_ref[...], preferred_element_type=jnp.float32)
    a = jax.nn.relu(a + b0_ref[...])
    out_ref[...] = (
        jnp.dot(w1t_ref[...], a, preferred_element_type=jnp.float32) + b1_ref[...]
    )


def _enc_tc(xT, w0t, b0c, w1t, b1c):
    grid = NP // CN
    return pl.pallas_call(
        _enc_body,
        grid=(grid,),
        in_specs=[
            pl.BlockSpec((128, CN), lambda i: (0, i)),
            pl.BlockSpec((H, 128), lambda i: (0, 0)),
            pl.BlockSpec((H, 1), lambda i: (0, 0)),
            pl.BlockSpec((H, H), lambda i: (0, 0)),
            pl.BlockSpec((H, 1), lambda i: (0, 0)),
        ],
        out_specs=pl.BlockSpec((H, CN), lambda i: (0, i)),
        out_shape=jax.ShapeDtypeStruct((H, NP), jnp.float32),
    )(xT, w0t, b0c, w1t, b1c)


def _proj_body(hT_ref, wab_ref, out_ref):
    out_ref[...] = jnp.dot(
        wab_ref[...], hT_ref[...], preferred_element_type=jnp.float32
    )


def _proj_tc(hT, wabT):
    grid = NP // CN
    return pl.pallas_call(
        _proj_body,
        grid=(grid,),
        in_specs=[
            pl.BlockSpec((H, CN), lambda i: (0, i)),
            pl.BlockSpec((2 * H, H), lambda i: (0, 0)),
        ],
        out_specs=pl.BlockSpec((2 * H, CN), lambda i: (0, i)),
        out_shape=jax.ShapeDtypeStruct((2 * H, NP), jnp.float32),
    )(hT, wabT)


def _mlp_body(zg_ref, rel_ref, wct_ref, b0_ref, w1t_ref, b1_ref, out_ref):
    z = zg_ref[...] + jnp.dot(
        wct_ref[...], rel_ref[...], preferred_element_type=jnp.float32
    ) + b0_ref[...]
    out_ref[...] = (
        jnp.dot(w1t_ref[...], jax.nn.relu(z), preferred_element_type=jnp.float32)
        + b1_ref[...]
    )


def _mlp_tc(zgT, relT, wcT8, b0c, w1t, b1c):
    grid = E // KT
    return pl.pallas_call(
        _mlp_body,
        grid=(grid,),
        in_specs=[
            pl.BlockSpec((H, KT), lambda i: (0, i)),
            pl.BlockSpec((8, KT), lambda i: (0, i)),
            pl.BlockSpec((H, 8), lambda i: (0, 0)),
            pl.BlockSpec((H, 1), lambda i: (0, 0)),
            pl.BlockSpec((H, H), lambda i: (0, 0)),
            pl.BlockSpec((H, 1), lambda i: (0, 0)),
        ],
        out_specs=pl.BlockSpec((H, KT), lambda i: (0, i)),
        out_shape=jax.ShapeDtypeStruct((H, E), jnp.float32),
    )(zgT, relT, wcT8, b0c, w1t, b1c)


def _pool_body(hT_ref, batch_ref, hw0t_ref, hb0_ref, hw1t_ref, hb1_ref, out_ref):
    hT = hT_ref[...]
    b = batch_ref[...]
    cols = []
    for g in range(G):
        msk = b == g
        col = jnp.max(jnp.where(msk, hT, _NEG_INF), axis=1, keepdims=True)
        cols.append(col)
    hg = jnp.concatenate(cols, axis=1)
    hg = jnp.where(hg == _NEG_INF, jnp.float32(0.0), hg)
    t = jax.nn.relu(
        jnp.dot(hw0t_ref[...], hg, preferred_element_type=jnp.float32)
        + hb0_ref[...]
    )
    out_ref[...] = (
        jnp.dot(hw1t_ref[...], t, preferred_element_type=jnp.float32)
        + hb1_ref[...]
    )


def _pool_tc(hT, batch2d, hw0t, hb0c, hw1t, hb1c):
    return pl.pallas_call(
        _pool_body,
        in_specs=[
            pl.BlockSpec((H, NP), lambda: (0, 0)),
            pl.BlockSpec((1, NP), lambda: (0, 0)),
            pl.BlockSpec((H, H), lambda: (0, 0)),
            pl.BlockSpec((H, 1), lambda: (0, 0)),
            pl.BlockSpec((1, H), lambda: (0, 0)),
            pl.BlockSpec((1, 1), lambda: (0, 0)),
        ],
        out_specs=pl.BlockSpec((1, G), lambda: (0, 0)),
        out_shape=jax.ShapeDtypeStruct((1, G), jnp.float32),
    )(hT, batch2d, hw0t, hb0c, hw1t, hb1c)


def kernel(x, edge_index, pos, batch, enc_W0, enc_b0, enc_W1, enc_b1,
           conv_W0, conv_b0, conv_W1, conv_b1,
           head_W0, head_b0, head_W1, head_b1):
    src = edge_index[0]
    dst = edge_index[1]

    xT = jnp.pad(x.T, ((0, 0), (0, NP - N)))
    posT = jnp.pad(pos.T, ((0, 0), (0, NP - N))).reshape(-1)
    batch2d = jnp.pad(batch, (0, NP - N), constant_values=-1)[None, :]

    enc_W0T = enc_W0.T
    enc_W1T = enc_W1.T
    enc_b0c = enc_b0[:, None]
    enc_b1c = enc_b1[:, None]
    head_W0T = head_W0.T
    head_W1T = head_W1.T
    head_b0c = head_b0[:, None]
    head_b1c = head_b1[:, None]

    wabTs, wcT8s, b0cs, w1Ts, b1cs = [], [], [], [], []
    for l in range(L):
        W0 = conv_W0[l]
        WaT = (W0[:H] - W0[H:2 * H]).T
        WbT = W0[H:2 * H].T
        wabTs.append(jnp.concatenate([WaT, WbT], axis=0))
        wcT8s.append(jnp.pad(W0[2 * H:].T, ((0, 0), (0, 5))))
        b0cs.append(conv_b0[l][:, None])
        w1Ts.append(conv_W1[l].T)
        b1cs.append(conv_b1[l][:, None])

    relT = _rel_kernel(posT, src, dst)
    relT = relT.reshape(8, E)

    hT = _enc_tc(xT, enc_W0T, enc_b0c, enc_W1T, enc_b1c)

    for l in range(L):
        uvT = _proj_tc(hT, wabTs[l])
        zgT = _gather_kernel(uvT.reshape(-1), src, dst)
        mT = _mlp_tc(zgT.reshape(H, E), relT, wcT8s[l], b0cs[l],
                     w1Ts[l], b1cs[l])
        hT = _scatter_kernel(mT.reshape(-1), dst)
        hT = hT.reshape(H, NP)

    outT = _pool_tc(hT, batch2d, head_W0T, head_b0c, head_W1T, head_b1c)
    return outT.reshape(G, 1)


# speedup vs baseline: 1.0114x; 1.0114x over previous
import functools

import jax
import jax.numpy as jnp
from jax import lax
from jax.experimental import pallas as pl
from jax.experimental.pallas import tpu as pltpu
from jax.experimental.pallas import tpu_sc as plsc

N = 10000
NP = 10240
E = 320000
H = 64
L = 6
G = 16
NW = 32
CE = 8000
KT = 16000
CN = 2048
EB = E // NW

_NEG_INF = float("-inf")


def _mesh():
    return plsc.VectorSubcoreMesh(core_axis_name="c", subcore_axis_name="s")


def _wid():
    return lax.axis_index("s") * 2 + lax.axis_index("c")


@functools.partial(
    pl.kernel, mesh=_mesh(),
    compiler_params=pltpu.CompilerParams(needs_layout_passes=False),
    out_type=jax.ShapeDtypeStruct((8 * E,), jnp.float32),
    scratch_types=[
        pltpu.VMEM((NP,), jnp.float32), pltpu.VMEM((NP,), jnp.float32),
        pltpu.VMEM((NP,), jnp.float32),
        pltpu.VMEM((EB,), jnp.int32), pltpu.VMEM((EB,), jnp.float32),
        pltpu.VMEM((EB,), jnp.float32), pltpu.VMEM((EB,), jnp.float32),
        pltpu.SemaphoreType.DMA, pltpu.SemaphoreType.DMA,
    ],
)
def _rel_kernel(posT_hbm, pk_hbm, out_hbm,
                px, py, pz, pkb, rx, ry, rz, xsem, osem):
    w = _wid()
    base = w * EB
    pltpu.sync_copy(posT_hbm.at[pl.ds(0, NP)], px)
    pltpu.sync_copy(posT_hbm.at[pl.ds(NP, NP)], py)
    pltpu.sync_copy(posT_hbm.at[pl.ds(2 * NP, NP)], pz)
    pltpu.sync_copy(pk_hbm.at[pl.ds(base, EB)], pkb)

    def body(g, _):
        sl = pl.ds(g * 16, 16)
        pk = pkb[sl]
        di = lax.shift_right_logical(pk, 14)
        si = lax.bitwise_and(pk, 16383)
        rx[sl] = plsc.load_gather(px, [si]) - plsc.load_gather(px, [di])
        ry[sl] = plsc.load_gather(py, [si]) - plsc.load_gather(py, [di])
        rz[sl] = plsc.load_gather(pz, [si]) - plsc.load_gather(pz, [di])
        return 0

    lax.fori_loop(0, EB // 16, body, 0)
    pltpu.async_copy(rx, out_hbm.at[pl.ds(0 * E + base, EB)], xsem)
    pltpu.async_copy(ry, out_hbm.at[pl.ds(1 * E + base, EB)], osem)
    pltpu.async_copy(rz, out_hbm.at[pl.ds(2 * E + base, EB)], osem)
    pltpu.make_async_copy(rx, out_hbm.at[pl.ds(0 * E + base, EB)], xsem).wait()

    def zbody(i, _):
        rx[pl.ds(i * 16, 16)] = jnp.zeros((16,), jnp.float32)
        return 0

    lax.fori_loop(0, EB // 16, zbody, 0)
    for r in range(3, 8):
        pltpu.async_copy(rx, out_hbm.at[pl.ds(r * E + base, EB)], osem)
    for r in range(3, 8):
        pltpu.make_async_copy(
            rx, out_hbm.at[pl.ds(r * E + base, EB)], osem).wait()
    pltpu.make_async_copy(ry, out_hbm.at[pl.ds(1 * E + base, EB)], osem).wait()
    pltpu.make_async_copy(rz, out_hbm.at[pl.ds(2 * E + base, EB)], osem).wait()


@functools.partial(
    pl.kernel, mesh=_mesh(),
    compiler_params=pltpu.CompilerParams(needs_layout_passes=False),
    out_type=jax.ShapeDtypeStruct((H * E,), jnp.float32),
    scratch_types=[
        pltpu.VMEM((NP,), jnp.float32), pltpu.VMEM((NP,), jnp.float32),
        pltpu.VMEM((NP,), jnp.float32), pltpu.VMEM((NP,), jnp.float32),
        pltpu.VMEM((CE,), jnp.int32), pltpu.VMEM((CE,), jnp.int32),
        pltpu.VMEM((CE,), jnp.float32), pltpu.VMEM((CE,), jnp.float32),
        pltpu.VMEM((CE,), jnp.float32), pltpu.VMEM((CE,), jnp.float32),
        pltpu.SemaphoreType.DMA, pltpu.SemaphoreType.DMA,
        pltpu.SemaphoreType.DMA, pltpu.SemaphoreType.DMA,
    ],
)
def _gather_kernel(uvT_hbm, pk_hbm, out_hbm,
                   u0, u1, v0, v1, pkb0, pkb1,
                   z00, z10, z01, z11, isem0, isem1, osem0, osem1):
    w = _wid()
    f0 = 2 * w
    pkbs = (pkb0, pkb1)
    z0s = (z00, z01)
    z1s = (z10, z11)
    isems = (isem0, isem1)
    osems = (osem0, osem1)
    NC = E // CE

    for b in range(2):
        pltpu.async_copy(pk_hbm.at[pl.ds(b * CE, CE)], pkbs[b], isems[b])
    pltpu.sync_copy(uvT_hbm.at[pl.ds(f0 * NP, NP)], u0)
    pltpu.sync_copy(uvT_hbm.at[pl.ds((f0 + 1) * NP, NP)], u1)
    pltpu.sync_copy(uvT_hbm.at[pl.ds((H + f0) * NP, NP)], v0)
    pltpu.sync_copy(uvT_hbm.at[pl.ds((H + f0 + 1) * NP, NP)], v1)

    def outer(i, _):
        for b in range(2):
            c = i * 2 + b
            eb = c * CE
            pkb, z0, z1 = pkbs[b], z0s[b], z1s[b]
            pltpu.make_async_copy(pk_hbm.at[pl.ds(eb, CE)], pkb,
                                  isems[b]).wait()

            @pl.when(c >= 2)
            def _():
                pltpu.make_async_copy(
                    z0, out_hbm.at[pl.ds(f0 * E + eb, CE)], osems[b]).wait()
                pltpu.make_async_copy(
                    z1, out_hbm.at[pl.ds((f0 + 1) * E + eb, CE)],
                    osems[b]).wait()

            def body(g, _):
                for k in range(4):
                    sl = pl.ds(g * 64 + k * 16, 16)
                    pk = pkb[sl]
                    di = lax.shift_right_logical(pk, 14)
                    si = lax.bitwise_and(pk, 16383)
                    z0[sl] = (plsc.load_gather(u0, [di])
                              + plsc.load_gather(v0, [si]))
                    z1[sl] = (plsc.load_gather(u1, [di])
                              + plsc.load_gather(v1, [si]))
                return 0

            lax.fori_loop(0, CE // 64, body, 0)
            pltpu.async_copy(z0, out_hbm.at[pl.ds(f0 * E + eb, CE)], osems[b])
            pltpu.async_copy(z1, out_hbm.at[pl.ds((f0 + 1) * E + eb, CE)],
                             osems[b])

            @pl.when(c + 2 < NC)
            def _():
                eb2 = eb + 2 * CE
                pltpu.async_copy(pk_hbm.at[pl.ds(eb2, CE)], pkb, isems[b])
        return 0

    lax.fori_loop(0, NC // 2, outer, 0)
    for b in range(2):
        eb = (NC - 2 + b) * CE
        pltpu.make_async_copy(
            z0s[b], out_hbm.at[pl.ds(f0 * E + eb, CE)], osems[b]).wait()
        pltpu.make_async_copy(
            z1s[b], out_hbm.at[pl.ds((f0 + 1) * E + eb, CE)], osems[b]).wait()


@functools.partial(
    pl.kernel, mesh=_mesh(),
    compiler_params=pltpu.CompilerParams(needs_layout_passes=False),
    out_type=jax.ShapeDtypeStruct((H * NP,), jnp.float32),
    scratch_types=[
        pltpu.VMEM((NP,), jnp.float32), pltpu.VMEM((NP,), jnp.float32),
        pltpu.VMEM((CE,), jnp.int32), pltpu.VMEM((CE,), jnp.int32),
        pltpu.VMEM((CE,), jnp.float32), pltpu.VMEM((CE,), jnp.float32),
        pltpu.VMEM((CE,), jnp.float32), pltpu.VMEM((CE,), jnp.float32),
        pltpu.SemaphoreType.DMA, pltpu.SemaphoreType.DMA,
    ],
)
def _scatter_kernel(mT_hbm, dst_hbm, out_hbm, acc0, acc1,
                    dstb0, dstb1, m0b0, m1b0, m0b1, m1b1, isem0, isem1):
    w = _wid()
    f0 = 2 * w
    dstbs = (dstb0, dstb1)
    m0bs = (m0b0, m0b1)
    m1bs = (m1b0, m1b1)
    isems = (isem0, isem1)
    NC = E // CE

    for b in range(2):
        eb = b * CE
        pltpu.async_copy(dst_hbm.at[pl.ds(eb, CE)], dstbs[b], isems[b])
        pltpu.async_copy(mT_hbm.at[pl.ds(f0 * E + eb, CE)], m0bs[b], isems[b])
        pltpu.async_copy(mT_hbm.at[pl.ds((f0 + 1) * E + eb, CE)], m1bs[b],
                         isems[b])

    def init(i, _):
        sl = pl.ds(i * 16, 16)
        acc0[sl] = jnp.full((16,), _NEG_INF, jnp.float32)
        acc1[sl] = jnp.full((16,), _NEG_INF, jnp.float32)
        return 0

    lax.fori_loop(0, NP // 16, init, 0)

    def outer_sc(i, _):
      for b in range(2):
        c = i * 2 + b
        eb = c * CE
        dstb, m0b, m1b = dstbs[b], m0bs[b], m1bs[b]
        pltpu.make_async_copy(dst_hbm.at[pl.ds(eb, CE)], dstb, isems[b]).wait()
        pltpu.make_async_copy(mT_hbm.at[pl.ds(f0 * E + eb, CE)], m0b,
                              isems[b]).wait()
        pltpu.make_async_copy(mT_hbm.at[pl.ds((f0 + 1) * E + eb, CE)], m1b,
                              isems[b]).wait()

        def body(g, _):
            for k in range(2):
                sl = pl.ds(g * 32 + k * 16, 16)
                idx = dstb[sl]
                val0 = m0b[sl]
                val1 = m1b[sl]
                cur0 = plsc.load_gather(acc0, [idx])
                cur1 = plsc.load_gather(acc1, [idx])
                plsc.store_scatter(acc0, [idx], jnp.maximum(cur0, val0))
                plsc.store_scatter(acc1, [idx], jnp.maximum(cur1, val1))
                c0 = plsc.load_gather(acc0, [idx])
                c1 = plsc.load_gather(acc1, [idx])

                def w_cond(nd):
                    n0, n1 = nd
                    return jnp.sum((n0 | n1).astype(jnp.int32)) > 0

                def w_body(nd):
                    n0, n1 = nd
                    plsc.store_scatter(acc0, [idx], val0, mask=n0)
                    plsc.store_scatter(acc1, [idx], val1, mask=n1)
                    r0 = plsc.load_gather(acc0, [idx])
                    r1 = plsc.load_gather(acc1, [idx])
                    return (val0 > r0, val1 > r1)

                lax.while_loop(w_cond, w_body, (val0 > c0, val1 > c1))
            return 0

        lax.fori_loop(0, CE // 32, body, 0)

        @pl.when(c + 2 < NC)
        def _():
            eb2 = eb + 2 * CE
            pltpu.async_copy(dst_hbm.at[pl.ds(eb2, CE)], dstb, isems[b])
            pltpu.async_copy(mT_hbm.at[pl.ds(f0 * E + eb2, CE)], m0b,
                             isems[b])
            pltpu.async_copy(mT_hbm.at[pl.ds((f0 + 1) * E + eb2, CE)], m1b,
                             isems[b])
      return 0

    lax.fori_loop(0, NC // 2, outer_sc, 0)

    def fini(i, _):
        sl = pl.ds(i * 16, 16)
        a0 = acc0[sl]
        a1 = acc1[sl]
        acc0[sl] = jnp.where(a0 == _NEG_INF, jnp.float32(0.0), a0)
        acc1[sl] = jnp.where(a1 == _NEG_INF, jnp.float32(0.0), a1)
        return 0

    lax.fori_loop(0, NP // 16, fini, 0)
    pltpu.sync_copy(acc0, out_hbm.at[pl.ds(f0 * NP, NP)])
    pltpu.sync_copy(acc1, out_hbm.at[pl.ds((f0 + 1) * NP, NP)])


def _enc_body(xT_ref, w0t_ref, b0_ref, w1t_ref, b1_ref, out_ref):
    a = jnp.dot(w0t_ref[...], xT_ref[...], preferred_element_type=jnp.float32)
    a = jax.nn.relu(a + b0_ref[...])
    out_ref[...] = (
        jnp.dot(w1t_ref[...], a, preferred_element_type=jnp.float32) + b1_ref[...]
    )


def _enc_tc(xT, w0t, b0c, w1t, b1c):
    grid = NP // CN
    return pl.pallas_call(
        _enc_body,
        grid=(grid,),
        in_specs=[
            pl.BlockSpec((128, CN), lambda i: (0, i)),
            pl.BlockSpec((H, 128), lambda i: (0, 0)),
            pl.BlockSpec((H, 1), lambda i: (0, 0)),
            pl.BlockSpec((H, H), lambda i: (0, 0)),
            pl.BlockSpec((H, 1), lambda i: (0, 0)),
        ],
        out_specs=pl.BlockSpec((H, CN), lambda i: (0, i)),
        out_shape=jax.ShapeDtypeStruct((H, NP), jnp.float32),
    )(xT, w0t, b0c, w1t, b1c)


def _proj_body(hT_ref, wab_ref, out_ref):
    out_ref[...] = jnp.dot(
        wab_ref[...], hT_ref[...], preferred_element_type=jnp.float32
    )


def _proj_tc(hT, wabT):
    grid = NP // CN
    return pl.pallas_call(
        _proj_body,
        grid=(grid,),
        in_specs=[
            pl.BlockSpec((H, CN), lambda i: (0, i)),
            pl.BlockSpec((2 * H, H), lambda i: (0, 0)),
        ],
        out_specs=pl.BlockSpec((2 * H, CN), lambda i: (0, i)),
        out_shape=jax.ShapeDtypeStruct((2 * H, NP), jnp.float32),
    )(hT, wabT)


def _mlp_body(zg_ref, rel_ref, wct_ref, b0_ref, w1t_ref, b1_ref, out_ref):
    z = zg_ref[...] + jnp.dot(
        wct_ref[...], rel_ref[...], preferred_element_type=jnp.float32
    ) + b0_ref[...]
    out_ref[...] = (
        jnp.dot(w1t_ref[...], jax.nn.relu(z), preferred_element_type=jnp.float32)
        + b1_ref[...]
    )


def _mlp_tc(zgT, relT, wcT8, b0c, w1t, b1c):
    grid = E // KT
    return pl.pallas_call(
        _mlp_body,
        grid=(grid,),
        in_specs=[
            pl.BlockSpec((H, KT), lambda i: (0, i)),
            pl.BlockSpec((8, KT), lambda i: (0, i)),
            pl.BlockSpec((H, 8), lambda i: (0, 0)),
            pl.BlockSpec((H, 1), lambda i: (0, 0)),
            pl.BlockSpec((H, H), lambda i: (0, 0)),
            pl.BlockSpec((H, 1), lambda i: (0, 0)),
        ],
        out_specs=pl.BlockSpec((H, KT), lambda i: (0, i)),
        out_shape=jax.ShapeDtypeStruct((H, E), jnp.float32),
    )(zgT, relT, wcT8, b0c, w1t, b1c)


def _pool_body(hT_ref, batch_ref, hw0t_ref, hb0_ref, hw1t_ref, hb1_ref, out_ref):
    hT = hT_ref[...]
    b = batch_ref[...]
    cols = []
    for g in range(G):
        msk = b == g
        col = jnp.max(jnp.where(msk, hT, _NEG_INF), axis=1, keepdims=True)
        cols.append(col)
    hg = jnp.concatenate(cols, axis=1)
    hg = jnp.where(hg == _NEG_INF, jnp.float32(0.0), hg)
    t = jax.nn.relu(
        jnp.dot(hw0t_ref[...], hg, preferred_element_type=jnp.float32)
        + hb0_ref[...]
    )
    out_ref[...] = (
        jnp.dot(hw1t_ref[...], t, preferred_element_type=jnp.float32)
        + hb1_ref[...]
    )


def _pool_tc(hT, batch2d, hw0t, hb0c, hw1t, hb1c):
    return pl.pallas_call(
        _pool_body,
        in_specs=[
            pl.BlockSpec((H, NP), lambda: (0, 0)),
            pl.BlockSpec((1, NP), lambda: (0, 0)),
            pl.BlockSpec((H, H), lambda: (0, 0)),
            pl.BlockSpec((H, 1), lambda: (0, 0)),
            pl.BlockSpec((1, H), lambda: (0, 0)),
            pl.BlockSpec((1, 1), lambda: (0, 0)),
        ],
        out_specs=pl.BlockSpec((1, G), lambda: (0, 0)),
        out_shape=jax.ShapeDtypeStruct((1, G), jnp.float32),
    )(hT, batch2d, hw0t, hb0c, hw1t, hb1c)


def kernel(x, edge_index, pos, batch, enc_W0, enc_b0, enc_W1, enc_b1,
           conv_W0, conv_b0, conv_W1, conv_b1,
           head_W0, head_b0, head_W1, head_b1):
    src = edge_index[0]
    dst = edge_index[1]
    pk = dst * 16384 + src

    xT = jnp.pad(x.T, ((0, 0), (0, NP - N)))
    posT = jnp.pad(pos.T, ((0, 0), (0, NP - N))).reshape(-1)
    batch2d = jnp.pad(batch, (0, NP - N), constant_values=-1)[None, :]

    enc_W0T = enc_W0.T
    enc_W1T = enc_W1.T
    enc_b0c = enc_b0[:, None]
    enc_b1c = enc_b1[:, None]
    head_W0T = head_W0.T
    head_W1T = head_W1.T
    head_b0c = head_b0[:, None]
    head_b1c = head_b1[:, None]

    wabTs, wcT8s, b0cs, w1Ts, b1cs = [], [], [], [], []
    for l in range(L):
        W0 = conv_W0[l]
        WaT = (W0[:H] - W0[H:2 * H]).T
        WbT = W0[H:2 * H].T
        wabTs.append(jnp.concatenate([WaT, WbT], axis=0))
        wcT8s.append(jnp.pad(W0[2 * H:].T, ((0, 0), (0, 5))))
        b0cs.append(conv_b0[l][:, None])
        w1Ts.append(conv_W1[l].T)
        b1cs.append(conv_b1[l][:, None])

    relT = _rel_kernel(posT, pk)
    relT = relT.reshape(8, E)

    hT = _enc_tc(xT, enc_W0T, enc_b0c, enc_W1T, enc_b1c)

    for l in range(L):
        uvT = _proj_tc(hT, wabTs[l])
        zgT = _gather_kernel(uvT.reshape(-1), pk)
        mT = _mlp_tc(zgT.reshape(H, E), relT, wcT8s[l], b0cs[l],
                     w1Ts[l], b1cs[l])
        hT = _scatter_kernel(mT.reshape(-1), dst)
        hT = hT.reshape(H, NP)

    outT = _pool_tc(hT, batch2d, head_W0T, head_b0c, head_W1T, head_b1c)
    return outT.reshape(G, 1)
